# baseline (speedup 1.0000x reference)
import functools

import jax
import jax.numpy as jnp
from jax import lax
from jax.experimental import pallas as pl
from jax.experimental.pallas import tpu as pltpu
from jax.experimental.pallas import tpu_sc as plsc

BATCH = 16384
EMBED_DIM = 64

N_SCS = 2
BPC = BATCH // N_SCS
ICH = 2048

_sc_mesh = plsc.ScalarSubcoreMesh(axis_name="core", num_cores=N_SCS)


@jax.jit
def _sc_gather(user_table, product_table, uidx, pidx):
    @functools.partial(
        pl.kernel,
        mesh=_sc_mesh,
        out_type=(
            jax.ShapeDtypeStruct((BATCH, EMBED_DIM), jnp.float32),
            jax.ShapeDtypeStruct((BATCH, EMBED_DIM), jnp.float32),
        ),
        scratch_types=[
            pltpu.SMEM((ICH,), jnp.int32),
            pltpu.SMEM((ICH,), jnp.int32),
            pltpu.SemaphoreType.DMA,
            pltpu.SemaphoreType.DMA,
            pltpu.SemaphoreType.DMA,
        ],
    )
    def k(ut_hbm, pt_hbm, ui_hbm, pi_hbm, ue_hbm, pe_hbm,
          ui_s, pi_s, isem, usem, psem):
        cid = lax.axis_index("core")
        base = cid * BPC
        for ch in range(BPC // ICH):
            cbase = base + ch * ICH
            ci = pltpu.async_copy(ui_hbm.at[pl.ds(cbase, ICH)], ui_s, isem)
            cp = pltpu.async_copy(pi_hbm.at[pl.ds(cbase, ICH)], pi_s, isem)
            ci.wait()
            cp.wait()

            @pl.loop(0, ICH)
            def _(i):
                r = ui_s[i]
                pltpu.make_async_copy(
                    ut_hbm.at[pl.ds(r, 1)], ue_hbm.at[pl.ds(cbase + i, 1)],
                    usem).start()
                q = pi_s[i]
                pltpu.make_async_copy(
                    pt_hbm.at[pl.ds(q, 1)], pe_hbm.at[pl.ds(cbase + i, 1)],
                    psem).start()

        pltpu.make_async_copy(
            ut_hbm.at[pl.ds(0, BPC)], ue_hbm.at[pl.ds(base, BPC)], usem).wait()
        pltpu.make_async_copy(
            pt_hbm.at[pl.ds(0, BPC)], pe_hbm.at[pl.ds(base, BPC)], psem).wait()

    return k(user_table, product_table, uidx, pidx)


def _mlp_body(ue, pe, uf, bd, w1, b1, w2, b2, w3a, w3b, w3c, b3, w4, b4, out):
    m = ue[...] * pe[...]
    ufeat = jnp.maximum(
        jnp.dot(uf[...], w1[...], preferred_element_type=jnp.float32) + b1[...], 0.0)
    bfeat = jnp.maximum(
        jnp.dot(bd[...], w2[...], preferred_element_type=jnp.float32) + b2[...], 0.0)
    h = (jnp.dot(m, w3a[...], preferred_element_type=jnp.float32)
         + jnp.dot(ufeat, w3b[...], preferred_element_type=jnp.float32)
         + jnp.dot(bfeat, w3c[...], preferred_element_type=jnp.float32)
         + b3[...])
    h = jnp.maximum(h, 0.0)
    logit = jnp.dot(h, w4[...], preferred_element_type=jnp.float32) + b4[...]
    out[...] = jax.nn.sigmoid(logit)


_TC_BLOCK = 2048


@jax.jit
def _tc_mlp(ue, pe, uf, bd, w1, b1, w2, b2, w3a, w3b, w3c, b3, w4, b4):
    def row_block(width):
        return pl.BlockSpec((_TC_BLOCK, width), lambda i: (i, 0))

    def whole(a):
        return pl.BlockSpec(a.shape, lambda i: (0, 0))

    return pl.pallas_call(
        _mlp_body,
        grid=(BATCH // _TC_BLOCK,),
        in_specs=[row_block(EMBED_DIM), row_block(EMBED_DIM), row_block(11),
                  row_block(3),
                  whole(w1), whole(b1), whole(w2), whole(b2),
                  whole(w3a), whole(w3b), whole(w3c), whole(b3),
                  whole(w4), whole(b4)],
        out_specs=row_block(1),
        out_shape=jax.ShapeDtypeStruct((BATCH, 1), jnp.float32),
    )(ue, pe, uf, bd, w1, b1, w2, b2, w3a, w3b, w3c, b3, w4, b4)


def kernel(user_ids, product_ids, user_features, behavior_data,
           user_table, product_table, W1, b1, W2, b2, W3, b3, W4, b4):
    ue, pe = _sc_gather(user_table, product_table, user_ids, product_ids)
    return _tc_mlp(
        ue, pe, user_features, behavior_data,
        W1.T, b1.reshape(1, 32), W2.T, b2.reshape(1, 32),
        W3[:, :EMBED_DIM].T, W3[:, EMBED_DIM:EMBED_DIM + 32].T,
        W3[:, EMBED_DIM + 32:].T, b3.reshape(1, 32),
        W4.T, b4.reshape(1, 1))
